# baseline (speedup 1.0000x reference)
import jax
import jax.numpy as jnp
from jax.experimental import pallas as pl

H, W, NPOS = 512, 512, 64
BH = 16


def _body(img_ref, hp_ref, vp_ref, hb_ref, vb_ref, hpos_ref, vpos_ref, out_ref):
    i = pl.program_id(0)
    img = img_ref[0]
    hp = hp_ref[...]
    hb = hb_ref[...]
    vp = vp_ref[...]
    vb = vb_ref[...]

    hpos = hpos_ref[...]
    vpos = vpos_ref[...]

    row_ids = jax.lax.broadcasted_iota(jnp.int32, (BH, 1, NPOS), 0) + i * BH
    hmask = jnp.any(row_ids == hpos, axis=2, keepdims=True)
    col_ids = jax.lax.broadcasted_iota(jnp.int32, (1, W, NPOS), 1)
    vmask = jnp.any(col_ids == vpos, axis=2, keepdims=True)
    grid = jnp.maximum(hmask.astype(jnp.float32), vmask.astype(jnp.float32))

    shp = (BH, W, 1)
    pieces = [
        img,
        jnp.broadcast_to(hp, shp),
        jnp.broadcast_to(vp, shp),
        jnp.broadcast_to(hb, shp),
        jnp.broadcast_to(vb, shp),
        jnp.broadcast_to(grid, shp),
    ]
    out_ref[0] = jnp.concatenate(pieces, axis=-1)


def kernel(normalized_image, h_probs, v_probs, h_binary, v_binary,
           h_positions, v_positions):
    hpos = h_positions.astype(jnp.int32).reshape(1, 1, NPOS)
    vpos = v_positions.astype(jnp.int32).reshape(1, 1, NPOS)
    out = pl.pallas_call(
        _body,
        grid=(H // BH,),
        in_specs=[
            pl.BlockSpec((1, BH, W, 3), lambda i: (0, i, 0, 0)),
            pl.BlockSpec((BH, 1, 1), lambda i: (i, 0, 0)),
            pl.BlockSpec((1, W, 1), lambda i: (0, 0, 0)),
            pl.BlockSpec((BH, 1, 1), lambda i: (i, 0, 0)),
            pl.BlockSpec((1, W, 1), lambda i: (0, 0, 0)),
            pl.BlockSpec((1, 1, NPOS), lambda i: (0, 0, 0)),
            pl.BlockSpec((1, 1, NPOS), lambda i: (0, 0, 0)),
        ],
        out_specs=pl.BlockSpec((1, BH, W, 8), lambda i: (0, i, 0, 0)),
        out_shape=jax.ShapeDtypeStruct((1, H, W, 8), jnp.float32),
    )(normalized_image, h_probs.reshape(H, 1, 1), v_probs.reshape(1, W, 1),
      h_binary.reshape(H, 1, 1), v_binary.reshape(1, W, 1), hpos, vpos)
    return out
